# baseline (speedup 1.0000x reference)
import jax
import jax.numpy as jnp
from jax.experimental import pallas as pl
from jax.experimental.pallas import tpu as pltpu
from jax.experimental.pallas import tpu_sc as plsc

HIDDEN = 768
INTER = 768
NUM_EXPERTS = 8
ALPHA = 1.702
LIMIT = 7.0
TOKENS = 2048
BT = 256
NB = TOKENS * 2 // BT + NUM_EXPERTS
P_PAD = NB * BT
CCH = 256

NC, NS, LANES = 2, 16, 16
NW = NC * NS
TPW = TOKENS // NW


def _router_body(x_ref, rw_ref, rb_ref, scores_ref, wout_ref, dest_ref,
                 binfo_ref):
    x = x_ref[...]
    rw = rw_ref[...]
    logits = jax.lax.dot_general(
        x, rw, (((1,), (1,)), ((), ())), preferred_element_type=jnp.float32
    ) + rb_ref[...]
    T, E = logits.shape
    eids = jax.lax.broadcasted_iota(jnp.int32, (T, E), 1)
    m1 = jnp.max(logits, axis=1, keepdims=True)
    i1 = jnp.min(jnp.where(logits == m1, eids, E), axis=1, keepdims=True)
    masked = jnp.where(eids == i1, -jnp.inf, logits)
    m2 = jnp.max(masked, axis=1, keepdims=True)
    i2 = jnp.min(jnp.where(masked == m2, eids, E), axis=1, keepdims=True)
    e2 = jnp.exp(m2 - m1)
    denom = 1.0 + e2
    p1 = 1.0 / denom
    p2 = e2 / denom
    scores_ref[...] = jnp.where(eids == i1, p1, 0.0) + jnp.where(eids == i2, p2, 0.0)
    lane0 = eids == 0
    lane1 = eids == 1
    wout_ref[...] = jnp.where(lane0, p1, 0.0) + jnp.where(lane1, p2, 0.0)

    onehot = (jnp.where(eids == i1, 1.0, 0.0) + jnp.where(eids == i2, 1.0, 0.0))
    counts = jnp.sum(onehot, axis=0, keepdims=True)
    padded = jnp.floor((counts + (BT - 1)) * (1.0 / BT)) * BT
    r8 = jax.lax.broadcasted_iota(jnp.int32, (E, E), 0)
    c8 = jax.lax.broadcasted_iota(jnp.int32, (E, E), 1)
    upper = jnp.where(r8 < c8, 1.0, 0.0).astype(jnp.bfloat16)
    start = jax.lax.dot_general(
        padded.astype(jnp.bfloat16), upper, (((1,), (0,)), ((), ())),
        preferred_element_type=jnp.float32)

    startblk = start * (1.0 / BT)
    bi = jax.lax.broadcasted_iota(jnp.int32, (NB, E), 0).astype(jnp.float32)
    nge = jnp.sum(jnp.where(bi >= startblk, 1.0, 0.0), axis=1, keepdims=True)
    binfo_ref[...] = jnp.broadcast_to(nge - 1.0, (NB, E)).astype(jnp.int32)

    rc = jax.lax.broadcasted_iota(jnp.int32, (CCH, CCH), 0)
    cc = jax.lax.broadcasted_iota(jnp.int32, (CCH, CCH), 1)
    ltri = jnp.where(rc > cc, 1.0, 0.0).astype(jnp.bfloat16)
    carry = jnp.zeros((1, E), jnp.float32)
    for c in range(T // CCH):
        oc = onehot[c * CCH:(c + 1) * CCH, :]
        rex = jax.lax.dot_general(
            ltri, oc.astype(jnp.bfloat16), (((1,), (0,)), ((), ())),
            preferred_element_type=jnp.float32) + carry
        carry = carry + jnp.sum(oc, axis=0, keepdims=True)
        pos = start + rex
        ec = eids[c * CCH:(c + 1) * CCH, :]
        i1c = i1[c * CCH:(c + 1) * CCH, :]
        i2c = i2[c * CCH:(c + 1) * CCH, :]
        d0 = jnp.sum(jnp.where(ec == i1c, pos, 0.0), axis=1, keepdims=True)
        d1 = jnp.sum(jnp.where(ec == i2c, pos, 0.0), axis=1, keepdims=True)
        dest_ref[pl.ds(c * CCH, CCH), :] = (
            jnp.where(lane0[c * CCH:(c + 1) * CCH, :], d0, 0.0)
            + jnp.where(lane1[c * CCH:(c + 1) * CCH, :], d1, 0.0)
        ).astype(jnp.int32)


def _dispatch_body(hs_hbm, d0_hbm, d1_hbm, xs_hbm,
                   hsbuf, d0buf, d1buf, sem):
    wid = jax.lax.axis_index("s") * NC + jax.lax.axis_index("c")
    base = wid * TPW
    pltpu.sync_copy(hs_hbm.at[pl.ds(base, TPW)], hsbuf)
    pltpu.sync_copy(d0_hbm.at[pl.ds(base, TPW)], d0buf)
    pltpu.sync_copy(d1_hbm.at[pl.ds(base, TPW)], d1buf)
    c0 = pltpu.async_copy(hsbuf, xs_hbm.at[d0buf], sem)
    c1 = pltpu.async_copy(hsbuf, xs_hbm.at[d1buf], sem)
    c0.wait()
    c1.wait()


def _expert_body(be_ref, x_ref, wg_ref, bg_ref, wu_ref, bu_ref,
                 wd_ref, bd_ref, y_ref):
    xc = x_ref[...].astype(jnp.bfloat16)
    g = jax.lax.dot_general(
        xc, wg_ref[0], (((1,), (0,)), ((), ())),
        preferred_element_type=jnp.float32) + bg_ref[0]
    u = jax.lax.dot_general(
        xc, wu_ref[0], (((1,), (0,)), ((), ())),
        preferred_element_type=jnp.float32) + bu_ref[0]
    g = jnp.minimum(g, LIMIT)
    u = jnp.clip(u, -LIMIT, LIMIT)
    glu = g * jax.nn.sigmoid(g * ALPHA)
    h = ((u + 1.0) * glu).astype(jnp.bfloat16)
    y = jax.lax.dot_general(
        h, wd_ref[0], (((1,), (0,)), ((), ())),
        preferred_element_type=jnp.float32) + bd_ref[0]
    y_ref[...] = y


def _combine_body(ys_hbm, d0_hbm, d1_hbm, w0_hbm, w1_hbm, out_hbm,
                  d0buf, d1buf, w0buf, w1buf, r0buf, r1buf, sem):
    wid = jax.lax.axis_index("s") * NC + jax.lax.axis_index("c")
    base = wid * TPW
    pltpu.sync_copy(d0_hbm.at[pl.ds(base, TPW)], d0buf)
    pltpu.sync_copy(d1_hbm.at[pl.ds(base, TPW)], d1buf)
    pltpu.sync_copy(w0_hbm.at[pl.ds(base, TPW)], w0buf)
    pltpu.sync_copy(w1_hbm.at[pl.ds(base, TPW)], w1buf)
    g0 = pltpu.async_copy(ys_hbm.at[d0buf], r0buf, sem)
    g1 = pltpu.async_copy(ys_hbm.at[d1buf], r1buf, sem)
    g0.wait()
    g1.wait()

    def body_j(j, _):
        w0 = w0buf[j, pl.ds(0, LANES)]
        w1 = w1buf[j, pl.ds(0, LANES)]

        def body_k(k, _):
            sl = pl.ds(k * LANES, LANES)
            r0buf[j, sl] = w0 * r0buf[j, sl] + w1 * r1buf[j, sl]
            return 0

        jax.lax.fori_loop(0, HIDDEN // LANES, body_k, 0)
        return 0

    jax.lax.fori_loop(0, TPW, body_j, 0)
    pltpu.sync_copy(r0buf, out_hbm.at[pl.ds(base, TPW)])


def kernel(hidden_states, router_weight, router_bias, gate_up_proj,
           gate_up_proj_bias, down_proj, down_proj_bias):
    B, S, H = hidden_states.shape
    T = B * S
    E = NUM_EXPERTS
    F = INTER
    hs = hidden_states.reshape(T, H)

    scores, wout, dest, binfo = pl.pallas_call(
        _router_body,
        out_shape=(
            jax.ShapeDtypeStruct((T, E), jnp.float32),
            jax.ShapeDtypeStruct((T, E), jnp.float32),
            jax.ShapeDtypeStruct((T, E), jnp.int32),
            jax.ShapeDtypeStruct((NB, E), jnp.int32),
        ),
    )(hs, router_weight, router_bias.reshape(1, E))

    d0 = dest[:, 0]
    d1 = dest[:, 1]
    w0 = jnp.broadcast_to(wout[:, 0:1], (T, 128))
    w1 = jnp.broadcast_to(wout[:, 1:2], (T, 128))

    mesh = plsc.VectorSubcoreMesh(core_axis_name="c", subcore_axis_name="s", num_cores=NC, num_subcores=NS)
    x_sorted = pl.kernel(
        _dispatch_body,
        out_type=jax.ShapeDtypeStruct((P_PAD, H), jnp.float32),
        mesh=mesh,
        scratch_types=[
            pltpu.VMEM((TPW, H), jnp.float32),
            pltpu.VMEM((TPW,), jnp.int32),
            pltpu.VMEM((TPW,), jnp.int32),
            pltpu.SemaphoreType.DMA,
        ],
    )(hs, d0, d1)

    wg = gate_up_proj[:, :, 0::2].astype(jnp.bfloat16)
    wu = gate_up_proj[:, :, 1::2].astype(jnp.bfloat16)
    wd16 = down_proj.astype(jnp.bfloat16)
    bg = gate_up_proj_bias[:, 0::2].reshape(E, 1, F)
    bu = gate_up_proj_bias[:, 1::2].reshape(E, 1, F)
    bd = down_proj_bias.reshape(E, 1, H)
    be = binfo[:, 0]

    y_sorted = pl.pallas_call(
        _expert_body,
        grid_spec=pltpu.PrefetchScalarGridSpec(
            num_scalar_prefetch=1,
            grid=(NB,),
            in_specs=[
                pl.BlockSpec((BT, H), lambda i, be_ref: (i, 0)),
                pl.BlockSpec((1, H, F), lambda i, be_ref: (be_ref[i], 0, 0)),
                pl.BlockSpec((1, 1, F), lambda i, be_ref: (be_ref[i], 0, 0)),
                pl.BlockSpec((1, H, F), lambda i, be_ref: (be_ref[i], 0, 0)),
                pl.BlockSpec((1, 1, F), lambda i, be_ref: (be_ref[i], 0, 0)),
                pl.BlockSpec((1, F, H), lambda i, be_ref: (be_ref[i], 0, 0)),
                pl.BlockSpec((1, 1, H), lambda i, be_ref: (be_ref[i], 0, 0)),
            ],
            out_specs=pl.BlockSpec((BT, H), lambda i, be_ref: (i, 0)),
        ),
        out_shape=jax.ShapeDtypeStruct((P_PAD, H), jnp.float32),
        compiler_params=pltpu.CompilerParams(
            dimension_semantics=("arbitrary",),
        ),
    )(be, x_sorted, wg, bg, wu, bu, wd16, bd)

    out = pl.kernel(
        _combine_body,
        out_type=jax.ShapeDtypeStruct((T, H), jnp.float32),
        mesh=plsc.VectorSubcoreMesh(core_axis_name="c", subcore_axis_name="s", num_cores=NC, num_subcores=NS),
        scratch_types=[
            pltpu.VMEM((TPW,), jnp.int32),
            pltpu.VMEM((TPW,), jnp.int32),
            pltpu.VMEM((TPW, 128), jnp.float32),
            pltpu.VMEM((TPW, 128), jnp.float32),
            pltpu.VMEM((TPW, H), jnp.float32),
            pltpu.VMEM((TPW, H), jnp.float32),
            pltpu.SemaphoreType.DMA,
        ],
    )(y_sorted, d0, d1, w0, w1)

    return out.reshape(B, S, H), scores


# speedup vs baseline: 1.0003x; 1.0003x over previous
import jax
import jax.numpy as jnp
from jax.experimental import pallas as pl
from jax.experimental.pallas import tpu as pltpu
from jax.experimental.pallas import tpu_sc as plsc

HIDDEN = 768
INTER = 768
NUM_EXPERTS = 8
ALPHA = 1.702
LIMIT = 7.0
TOKENS = 2048
BT = 256
NB = TOKENS * 2 // BT + NUM_EXPERTS
P_PAD = NB * BT
CCH = 256

NC, NS, LANES = 2, 16, 16
NW = NC * NS
TPW = TOKENS // NW


def _router_body(x_ref, rw_ref, rb_ref, scores_ref, wout_ref, dest_ref,
                 binfo_ref):
    x = x_ref[...]
    rw = rw_ref[...]
    logits = jax.lax.dot_general(
        x, rw, (((1,), (1,)), ((), ())), preferred_element_type=jnp.float32
    ) + rb_ref[...]
    T, E = logits.shape
    eids = jax.lax.broadcasted_iota(jnp.int32, (T, E), 1)
    m1 = jnp.max(logits, axis=1, keepdims=True)
    i1 = jnp.min(jnp.where(logits == m1, eids, E), axis=1, keepdims=True)
    masked = jnp.where(eids == i1, -jnp.inf, logits)
    m2 = jnp.max(masked, axis=1, keepdims=True)
    i2 = jnp.min(jnp.where(masked == m2, eids, E), axis=1, keepdims=True)
    e2 = jnp.exp(m2 - m1)
    denom = 1.0 + e2
    p1 = 1.0 / denom
    p2 = e2 / denom
    scores_ref[...] = jnp.where(eids == i1, p1, 0.0) + jnp.where(eids == i2, p2, 0.0)
    lane0 = eids == 0
    lane1 = eids == 1
    wout_ref[...] = jnp.where(lane0, p1, 0.0) + jnp.where(lane1, p2, 0.0)

    onehot = (jnp.where(eids == i1, 1.0, 0.0) + jnp.where(eids == i2, 1.0, 0.0))
    counts = jnp.sum(onehot, axis=0, keepdims=True)
    padded = jnp.floor((counts + (BT - 1)) * (1.0 / BT)) * BT
    r8 = jax.lax.broadcasted_iota(jnp.int32, (E, E), 0)
    c8 = jax.lax.broadcasted_iota(jnp.int32, (E, E), 1)
    upper = jnp.where(r8 < c8, 1.0, 0.0).astype(jnp.bfloat16)
    start = jax.lax.dot_general(
        padded.astype(jnp.bfloat16), upper, (((1,), (0,)), ((), ())),
        preferred_element_type=jnp.float32)

    startblk = start * (1.0 / BT)
    bi = jax.lax.broadcasted_iota(jnp.int32, (NB, E), 0).astype(jnp.float32)
    nge = jnp.sum(jnp.where(bi >= startblk, 1.0, 0.0), axis=1, keepdims=True)
    binfo_ref[...] = jnp.broadcast_to(nge - 1.0, (NB, E)).astype(jnp.int32)

    rc = jax.lax.broadcasted_iota(jnp.int32, (CCH, CCH), 0)
    cc = jax.lax.broadcasted_iota(jnp.int32, (CCH, CCH), 1)
    ltri = jnp.where(rc > cc, 1.0, 0.0).astype(jnp.bfloat16)
    carry = jnp.zeros((1, E), jnp.float32)
    for c in range(T // CCH):
        oc = onehot[c * CCH:(c + 1) * CCH, :]
        rex = jax.lax.dot_general(
            ltri, oc.astype(jnp.bfloat16), (((1,), (0,)), ((), ())),
            preferred_element_type=jnp.float32) + carry
        carry = carry + jnp.sum(oc, axis=0, keepdims=True)
        pos = start + rex
        ec = eids[c * CCH:(c + 1) * CCH, :]
        i1c = i1[c * CCH:(c + 1) * CCH, :]
        i2c = i2[c * CCH:(c + 1) * CCH, :]
        d0 = jnp.sum(jnp.where(ec == i1c, pos, 0.0), axis=1, keepdims=True)
        d1 = jnp.sum(jnp.where(ec == i2c, pos, 0.0), axis=1, keepdims=True)
        dest_ref[pl.ds(c * CCH, CCH), :] = (
            jnp.where(lane0[c * CCH:(c + 1) * CCH, :], d0, 0.0)
            + jnp.where(lane1[c * CCH:(c + 1) * CCH, :], d1, 0.0)
        ).astype(jnp.int32)


def _dispatch_body(hs_hbm, d0_hbm, d1_hbm, xs_hbm,
                   hsbuf, d0buf, d1buf, sem):
    wid = jax.lax.axis_index("s") * NC + jax.lax.axis_index("c")
    base = wid * TPW
    pltpu.sync_copy(hs_hbm.at[pl.ds(base, TPW)], hsbuf)
    pltpu.sync_copy(d0_hbm.at[pl.ds(base, TPW)], d0buf)
    pltpu.sync_copy(d1_hbm.at[pl.ds(base, TPW)], d1buf)
    c0 = pltpu.async_copy(hsbuf, xs_hbm.at[d0buf], sem)
    c1 = pltpu.async_copy(hsbuf, xs_hbm.at[d1buf], sem)
    c0.wait()
    c1.wait()


def _expert_body(be_ref, x_ref, wg_ref, wu_ref, bg_ref, bu_ref,
                 wd_ref, bd_ref, y_ref):
    xc = x_ref[...].astype(jnp.bfloat16)
    g = jax.lax.dot_general(
        xc, wg_ref[0], (((1,), (0,)), ((), ())),
        preferred_element_type=jnp.float32) + bg_ref[0]
    u = jax.lax.dot_general(
        xc, wu_ref[0], (((1,), (0,)), ((), ())),
        preferred_element_type=jnp.float32) + bu_ref[0]
    g = jnp.minimum(g, LIMIT)
    u = jnp.clip(u, -LIMIT, LIMIT)
    glu = g * jax.nn.sigmoid(g * ALPHA)
    h = ((u + 1.0) * glu).astype(jnp.bfloat16)
    y = jax.lax.dot_general(
        h, wd_ref[0], (((1,), (0,)), ((), ())),
        preferred_element_type=jnp.float32) + bd_ref[0]
    y_ref[...] = y


def _combine_body(ys_hbm, d0_hbm, d1_hbm, w0_hbm, w1_hbm, out_hbm,
                  d0buf, d1buf, w0buf, w1buf, r0buf, r1buf, sem):
    wid = jax.lax.axis_index("s") * NC + jax.lax.axis_index("c")
    base = wid * TPW
    pltpu.sync_copy(d0_hbm.at[pl.ds(base, TPW)], d0buf)
    pltpu.sync_copy(d1_hbm.at[pl.ds(base, TPW)], d1buf)
    pltpu.sync_copy(w0_hbm.at[pl.ds(base, TPW)], w0buf)
    pltpu.sync_copy(w1_hbm.at[pl.ds(base, TPW)], w1buf)
    g0 = pltpu.async_copy(ys_hbm.at[d0buf], r0buf, sem)
    g1 = pltpu.async_copy(ys_hbm.at[d1buf], r1buf, sem)
    g0.wait()
    g1.wait()

    def body_j(j, _):
        w0 = w0buf[j, pl.ds(0, LANES)]
        w1 = w1buf[j, pl.ds(0, LANES)]

        def body_k(k, _):
            sl = pl.ds(k * LANES, LANES)
            r0buf[j, sl] = w0 * r0buf[j, sl] + w1 * r1buf[j, sl]
            return 0

        jax.lax.fori_loop(0, HIDDEN // LANES, body_k, 0)
        return 0

    jax.lax.fori_loop(0, TPW, body_j, 0)
    pltpu.sync_copy(r0buf, out_hbm.at[pl.ds(base, TPW)])


def kernel(hidden_states, router_weight, router_bias, gate_up_proj,
           gate_up_proj_bias, down_proj, down_proj_bias):
    B, S, H = hidden_states.shape
    T = B * S
    E = NUM_EXPERTS
    F = INTER
    hs = hidden_states.reshape(T, H)

    scores, wout, dest, binfo = pl.pallas_call(
        _router_body,
        out_shape=(
            jax.ShapeDtypeStruct((T, E), jnp.float32),
            jax.ShapeDtypeStruct((T, E), jnp.float32),
            jax.ShapeDtypeStruct((T, E), jnp.int32),
            jax.ShapeDtypeStruct((NB, E), jnp.int32),
        ),
    )(hs, router_weight, router_bias.reshape(1, E))

    d0 = dest[:, 0]
    d1 = dest[:, 1]
    w0 = jnp.broadcast_to(wout[:, 0:1], (T, 128))
    w1 = jnp.broadcast_to(wout[:, 1:2], (T, 128))

    mesh = plsc.VectorSubcoreMesh(core_axis_name="c", subcore_axis_name="s", num_cores=NC, num_subcores=NS)
    x_sorted = pl.kernel(
        _dispatch_body,
        out_type=jax.ShapeDtypeStruct((P_PAD, H), jnp.float32),
        mesh=mesh,
        scratch_types=[
            pltpu.VMEM((TPW, H), jnp.float32),
            pltpu.VMEM((TPW,), jnp.int32),
            pltpu.VMEM((TPW,), jnp.int32),
            pltpu.SemaphoreType.DMA,
        ],
    )(hs, d0, d1)

    wg = gate_up_proj[:, :, 0::2].astype(jnp.bfloat16)
    wu = gate_up_proj[:, :, 1::2].astype(jnp.bfloat16)
    wd16 = down_proj.astype(jnp.bfloat16)
    bg = gate_up_proj_bias[:, 0::2].reshape(E, 1, F)
    bu = gate_up_proj_bias[:, 1::2].reshape(E, 1, F)
    bd = down_proj_bias.reshape(E, 1, H)
    be = binfo[:, 0]

    y_sorted = pl.pallas_call(
        _expert_body,
        grid_spec=pltpu.PrefetchScalarGridSpec(
            num_scalar_prefetch=1,
            grid=(NB,),
            in_specs=[
                pl.BlockSpec((BT, H), lambda i, be_ref: (i, 0)),
                pl.BlockSpec((1, H, F), lambda i, be_ref: (be_ref[i], 0, 0)),
                pl.BlockSpec((1, H, F), lambda i, be_ref: (be_ref[i], 0, 0)),
                pl.BlockSpec((1, 1, F), lambda i, be_ref: (be_ref[i], 0, 0)),
                pl.BlockSpec((1, 1, F), lambda i, be_ref: (be_ref[i], 0, 0)),
                pl.BlockSpec((1, F, H), lambda i, be_ref: (be_ref[i], 0, 0)),
                pl.BlockSpec((1, 1, H), lambda i, be_ref: (be_ref[i], 0, 0)),
            ],
            out_specs=pl.BlockSpec((BT, H), lambda i, be_ref: (i, 0)),
        ),
        out_shape=jax.ShapeDtypeStruct((P_PAD, H), jnp.float32),
        compiler_params=pltpu.CompilerParams(
            dimension_semantics=("arbitrary",),
        ),
    )(be, x_sorted, wg, wu, bg, bu, wd16, bd)

    out = pl.kernel(
        _combine_body,
        out_type=jax.ShapeDtypeStruct((T, H), jnp.float32),
        mesh=plsc.VectorSubcoreMesh(core_axis_name="c", subcore_axis_name="s", num_cores=NC, num_subcores=NS),
        scratch_types=[
            pltpu.VMEM((TPW,), jnp.int32),
            pltpu.VMEM((TPW,), jnp.int32),
            pltpu.VMEM((TPW, 128), jnp.float32),
            pltpu.VMEM((TPW, 128), jnp.float32),
            pltpu.VMEM((TPW, H), jnp.float32),
            pltpu.VMEM((TPW, H), jnp.float32),
            pltpu.SemaphoreType.DMA,
        ],
    )(y_sorted, d0, d1, w0, w1)

    return out.reshape(B, S, H), scores
